# baseline (speedup 1.0000x reference)
import functools
import math

import jax
import jax.numpy as jnp
from jax import lax
from jax.experimental import pallas as pl
from jax.experimental.pallas import tpu as pltpu
from jax.experimental.pallas import tpu_sc as plsc

_NSLICES = 5
_CHUNK = 64


def _sc_gather_slice(tables, idxs, k_off, n_slice, tc_tiling):
    info = plsc.get_sparse_core_info()
    nw = info.num_cores * info.num_subcores
    per_w = n_slice // nw
    chunk = _CHUNK
    n_chunks = per_w // chunk
    assert per_w % chunk == 0 and n_slice % nw == 0
    dims = [t.shape[1] for t in tables]

    mesh = plsc.VectorSubcoreMesh(core_axis_name="c", subcore_axis_name="s")
    out_type = [jax.ShapeDtypeStruct((n_slice, d), jnp.float32) for d in dims]
    scratch = (
        [pltpu.VMEM((per_w,), jnp.int32) for _ in tables]
        + [pltpu.VMEM((chunk, d), jnp.float32) for d in dims]
        + [pltpu.SemaphoreType.DMA]
    )

    @functools.partial(
        pl.kernel,
        out_type=out_type,
        mesh=mesh,
        scratch_types=scratch,
        compiler_params=pltpu.CompilerParams(use_tc_tiling_on_sc=tc_tiling),
    )
    def k(*refs):
        nt = len(tables)
        tabs = refs[:nt]
        idx_h = refs[nt:2 * nt]
        outs = refs[2 * nt:3 * nt]
        idx_v = refs[3 * nt:4 * nt]
        bufs = refs[4 * nt:5 * nt]
        sem = refs[5 * nt]

        wid = lax.axis_index("s") * info.num_cores + lax.axis_index("c")
        base = wid * per_w
        for t in range(nt):
            pltpu.sync_copy(idx_h[t].at[pl.ds(k_off + base, per_w)], idx_v[t])

        def body(j, _):
            off = j * chunk
            cps = []
            for t in range(nt):
                cps.append(
                    pltpu.async_copy(
                        tabs[t].at[idx_v[t].at[pl.ds(off, chunk)]], bufs[t], sem
                    )
                )
            for c in cps:
                c.wait()
            for t in range(nt):
                pltpu.sync_copy(bufs[t], outs[t].at[pl.ds(base + off, chunk)])
            return 0

        lax.fori_loop(0, n_chunks, body, 0)

    return k(*tables, *idxs)


_COS_POLY = (0.9999999922898433, -19.739205553483565, 64.93917219630283,
             -85.45116501824775, 60.17622317114787, -26.000498056834275,
             6.575565932038976)


def _tc_combine_body(u_ref, s1_ref, s2_ref, s3_ref, dis_ref, x3_ref,
                     wu_ref, w1_ref, w2_ref, w3_ref, b1_ref,
                     w21_ref, b21_ref, cwt_ref, cb_ref, om_ref, ceb_ref,
                     out_ref):
    xf_ref = x3_ref[0]
    f32 = jnp.float32
    pre = lax.dot(u_ref[:].astype(jnp.bfloat16), wu_ref[:],
                  preferred_element_type=f32)
    pre += lax.dot(s1_ref[:].astype(jnp.bfloat16), w1_ref[:],
                   preferred_element_type=f32)
    pre += lax.dot(s2_ref[:].astype(jnp.bfloat16), w2_ref[:],
                   preferred_element_type=f32)
    pre += lax.dot(s3_ref[:].astype(jnp.bfloat16), w3_ref[:],
                   preferred_element_type=f32)
    pre += b1_ref[:]
    alpha = 1.6732632423543772848170429916717
    scale = 1.0507009873554804934193349852946
    h1 = scale * jnp.where(pre > 0, pre, alpha * (jnp.exp(pre) - 1.0))
    mu = lax.dot(h1.astype(jnp.bfloat16), w21_ref[:],
                 preferred_element_type=f32) + b21_ref[:]
    x5 = xf_ref[:, 5:6]
    x6 = xf_ref[:, 6:7]
    t = xf_ref[:, 7:8]
    del xf_ref
    conp = x5 * cwt_ref[0:1, :] + x6 * cwt_ref[1:2, :] + cb_ref[:]
    r = t * om_ref[:] + ceb_ref[:]
    r = r - jnp.floor(r + 0.5)
    usq = r * r
    enc = jnp.full_like(usq, _COS_POLY[6])
    for c in _COS_POLY[5::-1]:
        enc = enc * usq + c
    div = math.sqrt(1.0 / 768.0)
    out_ref[0] = dis_ref[:] + conp + mu + div * enc


def _combine_specs(nbatch, d, grid_off):
    tok = lambda w: pl.BlockSpec((nbatch, w), lambda i: (i, 0))
    xtok = pl.BlockSpec((1, nbatch, 8), lambda i: (grid_off + i, 0, 0))
    out = pl.BlockSpec((1, nbatch, d), lambda i: (grid_off + i, 0, 0))
    return tok, xtok, out


def _tc_combine_slice(prev, gathered, xf, weights, seqlen, nbatch, d, kslice):
    n_slice = gathered[0].shape[0]
    grid = n_slice // nbatch
    tok, xtok, out_spec = _combine_specs(nbatch, d, kslice * grid)
    full = lambda a: pl.BlockSpec(a.shape, lambda i: (0,) * a.ndim)

    in_specs = [tok(128), tok(64), tok(32), tok(16), tok(d), xtok] \
        + [full(w) for w in weights]
    body = _tc_combine_body
    args = tuple(gathered) + (xf,) + tuple(weights)
    aliases = {}
    if prev is not None:
        in_specs = [pl.BlockSpec(memory_space=pl.ANY)] + in_specs
        body = lambda p, *refs: _tc_combine_body(*refs)
        args = (prev,) + args
        aliases = {0: 0}
    return pl.pallas_call(
        body,
        grid=(grid,),
        in_specs=in_specs,
        out_specs=out_spec,
        out_shape=jax.ShapeDtypeStruct((seqlen, nbatch, d), jnp.float32),
        input_output_aliases=aliases,
    )(*args)


def kernel(x, emb_u, emb_s1, emb_s2, emb_s3, fc1_w, fc1_b, fc21_w, fc21_b,
           fc22_w, fc22_b, dis_emb, con_w, con_b, omega, ce_bias):
    b, l, _ = x.shape
    n = b * l
    d = dis_emb.shape[1]
    xf = lax.optimization_barrier(jnp.transpose(x, (1, 0, 2)))
    idxs = [xf[:, :, k].astype(jnp.int32).reshape(n) for k in range(5)]
    wide_tables = [emb_u, dis_emb]
    narrow_tables = [emb_s1, emb_s2, emb_s3]

    bf16 = jnp.bfloat16
    w1t = fc1_w.T.astype(bf16)
    wu, w1, w2, w3 = w1t[:128], w1t[128:192], w1t[192:224], w1t[224:240]
    w21t = fc21_w.T.astype(bf16)
    b1 = fc1_b.reshape(1, -1)
    b21 = fc21_b.reshape(1, -1)
    cwt = con_w.T
    cb = con_b.reshape(1, -1)
    inv2pi = 1.0 / (2.0 * math.pi)
    om = omega.reshape(1, -1) * inv2pi
    ceb = ce_bias.reshape(1, -1) * inv2pi
    weights = (wu, w1, w2, w3, b1, w21t, b21, cwt, cb, om, ceb)

    n_slice = n // _NSLICES
    out = None
    ni = idxs[1:4]
    for ks in range(_NSLICES):
        k_off = ks * n_slice
        u, dis = _sc_gather_slice(wide_tables, [idxs[0], idxs[4]],
                                  k_off, n_slice, True)
        i1, i2, i3, u, dis = lax.optimization_barrier((*ni, u, dis))
        s1, s2, s3 = _sc_gather_slice(narrow_tables, [i1, i2, i3],
                                      k_off, n_slice, False)
        out = _tc_combine_slice(out, (u, s1, s2, s3, dis), xf, weights,
                                l, b, d, ks)
    return jnp.transpose(out, (1, 0, 2))


# speedup vs baseline: 1.0552x; 1.0552x over previous
import functools
import math

import jax
import jax.numpy as jnp
from jax import lax
from jax.experimental import pallas as pl
from jax.experimental.pallas import tpu as pltpu
from jax.experimental.pallas import tpu_sc as plsc

_NSLICES = 5
_CHUNK = 64


def _sc_gather_slice(tables, idxs, k_off, n_slice, tc_tiling):
    info = plsc.get_sparse_core_info()
    nw = info.num_cores * info.num_subcores
    per_w = n_slice // nw
    chunk = _CHUNK
    n_chunks = per_w // chunk
    assert per_w % chunk == 0 and n_slice % nw == 0
    dims = [t.shape[1] for t in tables]

    mesh = plsc.VectorSubcoreMesh(core_axis_name="c", subcore_axis_name="s")
    out_type = [jax.ShapeDtypeStruct((n_slice, d), jnp.float32) for d in dims]
    scratch = (
        [pltpu.VMEM((per_w,), jnp.int32) for _ in tables]
        + [pltpu.VMEM((chunk, d), jnp.float32) for d in dims]
        + [pltpu.SemaphoreType.DMA]
    )

    @functools.partial(
        pl.kernel,
        out_type=out_type,
        mesh=mesh,
        scratch_types=scratch,
        compiler_params=pltpu.CompilerParams(use_tc_tiling_on_sc=tc_tiling),
    )
    def k(*refs):
        nt = len(tables)
        tabs = refs[:nt]
        idx_h = refs[nt:2 * nt]
        outs = refs[2 * nt:3 * nt]
        idx_v = refs[3 * nt:4 * nt]
        bufs = refs[4 * nt:5 * nt]
        sem = refs[5 * nt]

        wid = lax.axis_index("s") * info.num_cores + lax.axis_index("c")
        base = wid * per_w
        for t in range(nt):
            pltpu.sync_copy(idx_h[t].at[pl.ds(k_off + base, per_w)], idx_v[t])

        def body(j, _):
            off = j * chunk
            cps = []
            for t in range(nt):
                cps.append(
                    pltpu.async_copy(
                        tabs[t].at[idx_v[t].at[pl.ds(off, chunk)]], bufs[t], sem
                    )
                )
            for c in cps:
                c.wait()
            for t in range(nt):
                pltpu.sync_copy(bufs[t], outs[t].at[pl.ds(base + off, chunk)])
            return 0

        lax.fori_loop(0, n_chunks, body, 0)

    return k(*tables, *idxs)


_COS_POLY = (0.9999999922898433, -19.739205553483565, 64.93917219630283,
             -85.45116501824775, 60.17622317114787, -26.000498056834275,
             6.575565932038976)


def _tc_combine_body(u_ref, s1_ref, s2_ref, s3_ref, dis_ref, x3_ref,
                     wu_ref, w1_ref, w2_ref, w3_ref, b1_ref,
                     w21_ref, b21_ref, cwt_ref, cb_ref, om_ref, ceb_ref,
                     out_ref):
    xf_ref = x3_ref[0]
    f32 = jnp.float32
    pre = lax.dot(u_ref[:].astype(jnp.bfloat16), wu_ref[:],
                  preferred_element_type=f32)
    pre += lax.dot(s1_ref[:].astype(jnp.bfloat16), w1_ref[:],
                   preferred_element_type=f32)
    pre += lax.dot(s2_ref[:].astype(jnp.bfloat16), w2_ref[:],
                   preferred_element_type=f32)
    pre += lax.dot(s3_ref[:].astype(jnp.bfloat16), w3_ref[:],
                   preferred_element_type=f32)
    pre += b1_ref[:]
    alpha = 1.6732632423543772848170429916717
    scale = 1.0507009873554804934193349852946
    h1 = scale * jnp.where(pre > 0, pre, alpha * (jnp.exp(pre) - 1.0))
    mu = lax.dot(h1.astype(jnp.bfloat16), w21_ref[:],
                 preferred_element_type=f32) + b21_ref[:]
    x5 = xf_ref[:, 5:6]
    x6 = xf_ref[:, 6:7]
    t = xf_ref[:, 7:8]
    del xf_ref
    conp = x5 * cwt_ref[0:1, :] + x6 * cwt_ref[1:2, :] + cb_ref[:]
    r = t * om_ref[:] + ceb_ref[:]
    r = r - jnp.floor(r + 0.5)
    usq = r * r
    enc = jnp.full_like(usq, _COS_POLY[6])
    for c in _COS_POLY[5::-1]:
        enc = enc * usq + c
    div = math.sqrt(1.0 / 768.0)
    out_ref[0] = dis_ref[:] + conp + mu + div * enc


def _combine_specs(nbatch, d, grid_off):
    tok = lambda w: pl.BlockSpec((nbatch, w), lambda i: (i, 0))
    xtok = pl.BlockSpec((1, nbatch, 8), lambda i: (grid_off + i, 0, 0))
    out = pl.BlockSpec((1, nbatch, d), lambda i: (grid_off + i, 0, 0))
    return tok, xtok, out


def _tc_combine_slice(prev, gathered, xf, weights, seqlen, nbatch, d, kslice):
    n_slice = gathered[0].shape[0]
    grid = n_slice // nbatch
    tok, xtok, out_spec = _combine_specs(nbatch, d, kslice * grid)
    full = lambda a: pl.BlockSpec(a.shape, lambda i: (0,) * a.ndim)

    in_specs = [tok(128), tok(64), tok(32), tok(16), tok(d), xtok] \
        + [full(w) for w in weights]
    body = _tc_combine_body
    args = tuple(gathered) + (xf,) + tuple(weights)
    aliases = {}
    if prev is not None:
        in_specs = [pl.BlockSpec(memory_space=pl.ANY)] + in_specs
        body = lambda p, *refs: _tc_combine_body(*refs)
        args = (prev,) + args
        aliases = {0: 0}
    return pl.pallas_call(
        body,
        grid=(grid,),
        in_specs=in_specs,
        out_specs=out_spec,
        out_shape=jax.ShapeDtypeStruct((seqlen, nbatch, d), jnp.float32),
        input_output_aliases=aliases,
    )(*args)


def kernel(x, emb_u, emb_s1, emb_s2, emb_s3, fc1_w, fc1_b, fc21_w, fc21_b,
           fc22_w, fc22_b, dis_emb, con_w, con_b, omega, ce_bias):
    b, l, _ = x.shape
    n = b * l
    d = dis_emb.shape[1]
    xf = lax.optimization_barrier(jnp.transpose(x, (1, 0, 2)))
    idxs = [xf[:, :, k].astype(jnp.int32).reshape(n) for k in range(5)]
    wide_tables = [emb_u, dis_emb]
    narrow_tables = [emb_s1, emb_s2, emb_s3]

    bf16 = jnp.bfloat16
    w1t = fc1_w.T.astype(bf16)
    wu, w1, w2, w3 = w1t[:128], w1t[128:192], w1t[192:224], w1t[224:240]
    w21t = fc21_w.T.astype(bf16)
    b1 = fc1_b.reshape(1, -1)
    b21 = fc21_b.reshape(1, -1)
    cwt = con_w.T
    cb = con_b.reshape(1, -1)
    inv2pi = 1.0 / (2.0 * math.pi)
    om = omega.reshape(1, -1) * inv2pi
    ceb = ce_bias.reshape(1, -1) * inv2pi
    weights = (wu, w1, w2, w3, b1, w21t, b21, cwt, cb, om, ceb)

    n_slice = n // _NSLICES
    out = None
    ni = idxs[1:4]
    for ks in range(_NSLICES):
        k_off = ks * n_slice
        u, dis = _sc_gather_slice(wide_tables, [idxs[0], idxs[4]],
                                  k_off, n_slice, True)
        s1, s2, s3 = _sc_gather_slice(narrow_tables, ni,
                                      k_off, n_slice, False)
        out = _tc_combine_slice(out, (u, s1, s2, s3, dis), xf, weights,
                                l, b, d, ks)
    return jnp.transpose(out, (1, 0, 2))
